# baseline (speedup 1.0000x reference)
import functools

import jax
import jax.numpy as jnp
from jax import lax
from jax.experimental import pallas as pl
from jax.experimental.pallas import tpu as pltpu
from jax.experimental.pallas import tpu_sc as plsc

NUM_WORKERS = 32
CHUNK = 128
NB = 5
PF = 4


def _embed_body(n_chunks, chunk, d, n_groups, b0, idx_hbm, table_hbm,
                out_hbm, idx_v, rows_v, *sems):
    gsem = sems[:NB]
    ssem = sems[NB:2 * NB]
    isem = sems[2 * NB]
    wid = lax.axis_index("s") * 2 + lax.axis_index("c")
    col = wid * chunk

    def gather(j, b):
        pltpu.async_copy(table_hbm.at[idx_v.at[j]], rows_v.at[b], gsem[b])

    def wait_gather(b):
        pltpu.make_async_copy(table_hbm.at[pl.ds(0, chunk)], rows_v.at[b],
                              gsem[b]).wait()

    def scatter(j, b):
        pltpu.async_copy(rows_v.at[b],
                         out_hbm.at[pl.ds(j * b0 + col, chunk)], ssem[b])

    def wait_scatter(b):
        pltpu.make_async_copy(rows_v.at[b], out_hbm.at[pl.ds(col, chunk)],
                              ssem[b]).wait()

    pltpu.sync_copy(idx_hbm.at[pl.ds(0, 8), pl.ds(col, chunk)],
                    idx_v.at[pl.ds(0, 8)])

    for b in range(PF):
        gather(b, b)

    pltpu.async_copy(idx_hbm.at[:, pl.ds(col, chunk)], idx_v, isem)
    pltpu.make_async_copy(idx_hbm.at[:, pl.ds(col, chunk)], idx_v,
                          isem).wait()

    for b in range(NB):
        wait_gather(b)
        scatter(b, b)
        if b + PF < n_chunks:
            if b + PF - NB >= 0:
                wait_scatter((b + PF) % NB)
            gather(b + PF, (b + PF) % NB)

    def outer(g, carry):
        for b in range(NB):
            j = g * NB + b
            wait_gather(b)
            scatter(j, b)
            wait_scatter((b + PF) % NB)
            gather(j + PF, (b + PF) % NB)
        return carry

    lax.fori_loop(1, n_groups - 1, outer, 0)

    for b in range(NB):
        j = (n_groups - 1) * NB + b
        wait_gather(b)
        scatter(j, b)
        if j + PF < n_chunks:
            wait_scatter((b + PF) % NB)
            gather(j + PF, (b + PF) % NB)

    for b in range(NB):
        wait_scatter(b)


def kernel(ids, table):
    b0, seq = ids.shape
    d = table.shape[1]
    n_chunks = seq
    n_groups = n_chunks // NB
    idx = ids.T.astype(jnp.int32)

    mesh = plsc.VectorSubcoreMesh(core_axis_name="c", subcore_axis_name="s")
    embed = functools.partial(_embed_body, n_chunks, CHUNK, d, n_groups, b0)
    out = pl.kernel(
        embed,
        mesh=mesh,
        out_type=jax.ShapeDtypeStruct((b0 * seq, d), jnp.float32),
        scratch_types=[
            pltpu.VMEM((n_chunks, CHUNK), jnp.int32),
            pltpu.VMEM((NB, CHUNK, d), jnp.float32),
        ] + [pltpu.SemaphoreType.DMA] * (2 * NB + 1),
    )(idx, table)
    return out.reshape(seq, b0, d).transpose(1, 0, 2)


# speedup vs baseline: 1.0049x; 1.0049x over previous
import functools

import jax
import jax.numpy as jnp
from jax import lax
from jax.experimental import pallas as pl
from jax.experimental.pallas import tpu as pltpu
from jax.experimental.pallas import tpu_sc as plsc

NUM_WORKERS = 32
CHUNK = 128
NB = 5
PF = 4


def _embed_body(n_chunks, chunk, d, n_groups, b0, idx_hbm, table_hbm,
                out_hbm, idx_v, rows_v, *sems):
    gsem = sems[:NB]
    ssem = sems[NB:]
    wid = lax.axis_index("s") * 2 + lax.axis_index("c")
    col = wid * chunk

    def gather(j, b):
        pltpu.async_copy(table_hbm.at[idx_v.at[j]], rows_v.at[b], gsem[b])

    def wait_gather(b):
        pltpu.make_async_copy(table_hbm.at[pl.ds(0, chunk)], rows_v.at[b],
                              gsem[b]).wait()

    def scatter(j, b):
        pltpu.async_copy(rows_v.at[b],
                         out_hbm.at[pl.ds(j * b0 + col, chunk)], ssem[b])

    def wait_scatter(b):
        pltpu.make_async_copy(rows_v.at[b], out_hbm.at[pl.ds(col, chunk)],
                              ssem[b]).wait()

    pltpu.sync_copy(idx_hbm.at[:, pl.ds(col, chunk)], idx_v)

    for b in range(PF):
        gather(b, b)

    for b in range(NB):
        wait_gather(b)
        scatter(b, b)
        if b + PF < n_chunks:
            if b + PF - NB >= 0:
                wait_scatter((b + PF) % NB)
            gather(b + PF, (b + PF) % NB)

    def outer(g, carry):
        for b in range(NB):
            j = g * NB + b
            wait_gather(b)
            scatter(j, b)
            wait_scatter((b + PF) % NB)
            gather(j + PF, (b + PF) % NB)
        return carry

    lax.fori_loop(1, n_groups - 1, outer, 0)

    for b in range(NB):
        j = (n_groups - 1) * NB + b
        wait_gather(b)
        scatter(j, b)
        if j + PF < n_chunks:
            wait_scatter((b + PF) % NB)
            gather(j + PF, (b + PF) % NB)

    for b in range(NB):
        wait_scatter(b)


def kernel(ids, table):
    b0, seq = ids.shape
    d = table.shape[1]
    n_chunks = seq
    n_groups = n_chunks // NB
    idx = ids.T.astype(jnp.int32)

    mesh = plsc.VectorSubcoreMesh(core_axis_name="c", subcore_axis_name="s")
    embed = functools.partial(_embed_body, n_chunks, CHUNK, d, n_groups, b0)
    out = pl.kernel(
        embed,
        mesh=mesh,
        out_type=jax.ShapeDtypeStruct((b0 * seq, d), jnp.float32),
        scratch_types=[
            pltpu.VMEM((n_chunks, CHUNK), jnp.int32),
            pltpu.VMEM((NB, CHUNK, d), jnp.float32),
        ] + [pltpu.SemaphoreType.DMA] * (2 * NB),
    )(idx, table)
    return out.reshape(seq, b0, d).transpose(1, 0, 2)
